# baseline (speedup 1.0000x reference)
import functools

import jax
import jax.numpy as jnp
from jax import lax
from jax.experimental import pallas as pl
from jax.experimental.pallas import tpu as pltpu
from jax.experimental.pallas import tpu_sc as plsc

N_NODES = 10000
N_ACC = 10240
BLK = 400

NT = 16
EB = 128
NB = 162
EPT = NB * EB
E_PAD = NT * EPT
SB_LEN = 27
RPT = N_ACC // NT
DR = 128

_i32 = jnp.int32


def _make_cheb_layer(DH):
    mesh = plsc.VectorSubcoreMesh(core_axis_name="c", subcore_axis_name="s")
    oshape = jax.ShapeDtypeStruct((2, N_ACC, DH), jnp.float32)

    @functools.partial(
        pl.kernel,
        out_type=(oshape, oshape, oshape),
        mesh=mesh,
        compiler_params=pltpu.CompilerParams(use_tc_tiling_on_sc=False),
        scratch_types=[
            pltpu.VMEM((SB_LEN, EB), jnp.int32),
            pltpu.VMEM((SB_LEN, EB), jnp.int32),
            pltpu.VMEM((SB_LEN, EB), jnp.float32),
            pltpu.VMEM((EB, DH), jnp.float32),
            pltpu.VMEM((EB, DH), jnp.float32),
            pltpu.VMEM_SHARED((N_ACC, DH), jnp.float32),
            pltpu.SemaphoreType.DMA,
        ],
    )
    def layer(t0_hbm, src_hbm, dst_hbm, w_hbm, tx1_hbm, tx2_hbm, tx3_hbm,
              srcv, dstv, wv, rows, rows2, acc, sem):
        c = lax.axis_index("c")
        s = lax.axis_index("s")
        base = s * _i32(RPT)

        def zero_acc():
            def zrow(r, _):
                for ch in range(DH // 16):
                    rows[r, pl.ds(ch * 16, 16)] = jnp.zeros((16,), jnp.float32)
                return 0

            lax.fori_loop(_i32(0), _i32(EB), zrow, 0)
            for k in range(RPT // DR):
                pltpu.sync_copy(rows, acc.at[pl.ds(base + _i32(k * DR), DR)])

        def edge_sweep(t_hbm):
            def superstep(sb, _):
                off = sb * _i32(SB_LEN)
                pltpu.sync_copy(src_hbm.at[s, pl.ds(off, SB_LEN)], srcv)
                pltpu.sync_copy(dst_hbm.at[s, pl.ds(off, SB_LEN)], dstv)
                pltpu.sync_copy(w_hbm.at[s, pl.ds(off, SB_LEN)], wv)

                def step(g, _):
                    pltpu.async_copy(
                        t_hbm.at[c].at[srcv.at[g]], rows, sem
                    ).wait()

                    def sgrp(q, _):
                        w16 = wv[g, pl.ds(q * _i32(16), 16)]
                        for jj in range(16):
                            wj = w16[jj]
                            j = q * _i32(16) + _i32(jj)
                            for ch in range(DH // 16):
                                sl = pl.ds(ch * 16, 16)
                                rows[j, sl] = rows[j, sl] * wj
                        return 0

                    lax.fori_loop(_i32(0), _i32(EB // 16), sgrp, 0)
                    pltpu.sync_copy(rows, acc.at[dstv.at[g]], add=True)
                    return 0

                lax.fori_loop(_i32(0), _i32(SB_LEN), step, 0)
                return 0

            lax.fori_loop(_i32(0), _i32(NB // SB_LEN), superstep, 0)

        def drain(out_hbm, prev_hbm):
            for k in range(RPT // DR):
                st = base + _i32(k * DR)
                pltpu.sync_copy(acc.at[pl.ds(st, DR)], rows)
                if prev_hbm is not None:
                    pltpu.sync_copy(prev_hbm.at[c, pl.ds(st, DR)], rows2)

                    def crow(r, _):
                        for ch in range(DH // 16):
                            sl = pl.ds(ch * 16, 16)
                            rows[r, sl] = rows[r, sl] * 2.0 - rows2[r, sl]
                        return 0

                    lax.fori_loop(_i32(0), _i32(DR), crow, 0)
                pltpu.sync_copy(rows, out_hbm.at[c, pl.ds(st, DR)])

        def mv_pass(t_hbm, out_hbm, prev_hbm):
            zero_acc()
            plsc.subcore_barrier()
            edge_sweep(t_hbm)
            plsc.subcore_barrier()
            drain(out_hbm, prev_hbm)
            plsc.subcore_barrier()

        mv_pass(t0_hbm, tx1_hbm, None)
        mv_pass(tx1_hbm, tx2_hbm, t0_hbm)
        mv_pass(tx2_hbm, tx3_hbm, tx1_hbm)

    return layer


_CHEB = {dh: _make_cheb_layer(dh) for dh in (80, 64)}


def _cat(ref):
    return jnp.concatenate([ref[0], ref[1]], axis=1)


def _mm_relu_body(t0, t1, t2, t3, w_ref, b_ref, o_ref):
    tcat = jnp.concatenate([_cat(t0), _cat(t1), _cat(t2), _cat(t3)], axis=1)
    h = jax.nn.relu(
        jnp.dot(tcat, w_ref[...], preferred_element_type=jnp.float32) + b_ref[...]
    )
    hid2 = o_ref.shape[2]
    o_ref[0] = h[:, :hid2]
    o_ref[1] = h[:, hid2:]


def _mm_relu_stacked(txs, wcat, b):
    dh = txs[0].shape[2]
    kdim, hid = wcat.shape
    grid = N_NODES // BLK
    tspec = pl.BlockSpec((2, BLK, dh), lambda i: (i * 0, i, i * 0))
    return pl.pallas_call(
        _mm_relu_body,
        grid=(grid,),
        in_specs=[tspec, tspec, tspec, tspec,
                  pl.BlockSpec((kdim, hid), lambda i: (i * 0, i * 0)),
                  pl.BlockSpec((1, hid), lambda i: (i * 0, i * 0))],
        out_specs=pl.BlockSpec((2, BLK, hid // 2), lambda i: (i * 0, i, i * 0)),
        out_shape=jax.ShapeDtypeStruct((2, N_ACC, hid // 2), jnp.float32),
    )(*txs, wcat, b)


def _mm_relu_sum_body(t0, t1, t2, t3, w_ref, b_ref, o_ref):
    i = pl.program_id(0)
    tcat = jnp.concatenate([_cat(t0), _cat(t1), _cat(t2), _cat(t3)], axis=1)
    h = jax.nn.relu(
        jnp.dot(tcat, w_ref[...], preferred_element_type=jnp.float32) + b_ref[...]
    )
    part = jnp.sum(h, axis=0, keepdims=True)

    @pl.when(i == 0)
    def _():
        o_ref[...] = jnp.zeros_like(o_ref)

    o_ref[...] += part


def _mm_relu_sum(txs, wcat, b):
    dh = txs[0].shape[2]
    kdim, hid = wcat.shape
    grid = N_NODES // BLK
    tspec = pl.BlockSpec((2, BLK, dh), lambda i: (i * 0, i, i * 0))
    return pl.pallas_call(
        _mm_relu_sum_body,
        grid=(grid,),
        in_specs=[tspec, tspec, tspec, tspec,
                  pl.BlockSpec((kdim, hid), lambda i: (i * 0, i * 0)),
                  pl.BlockSpec((1, hid), lambda i: (i * 0, i * 0))],
        out_specs=pl.BlockSpec((1, hid), lambda i: (i * 0, i * 0)),
        out_shape=jax.ShapeDtypeStruct((1, hid), jnp.float32),
    )(*txs, wcat, b)


def _heads_body(s_ref, wmu_ref, bmu_ref, wlv_ref, blv_ref, mu_ref, lv_ref):
    ge = s_ref[...] * (1.0 / N_NODES)
    mu_ref[...] = (
        jnp.dot(ge, wmu_ref[...], preferred_element_type=jnp.float32) + bmu_ref[...]
    )
    lv_ref[...] = (
        jnp.dot(ge, wlv_ref[...], preferred_element_type=jnp.float32) + blv_ref[...]
    )


def _heads(hsum, Wmu, bmu, Wlv, blv):
    lat = Wmu.shape[1]
    return pl.pallas_call(
        _heads_body,
        out_shape=(
            jax.ShapeDtypeStruct((1, lat), jnp.float32),
            jax.ShapeDtypeStruct((1, lat), jnp.float32),
        ),
    )(hsum, Wmu, bmu.reshape(1, -1), Wlv, blv.reshape(1, -1))


def _stack_halves(a, dh):
    a = jnp.pad(a, ((0, N_ACC - a.shape[0]), (0, 2 * dh - a.shape[1])))
    return jnp.stack([a[:, :dh], a[:, dh:]])


def kernel(x, edge_index, lap_pe, edge_weight, W1, b1, W2, b2, Wmu, bmu, Wlv, blv):
    num_nodes = x.shape[0]
    src = edge_index[0].astype(jnp.int32)
    dst = edge_index[1].astype(jnp.int32)
    loop = jnp.arange(num_nodes, dtype=jnp.int32)
    src_e = jnp.concatenate([src, loop])
    dst_e = jnp.concatenate([dst, loop])
    w_e = jnp.concatenate([edge_weight, jnp.ones((num_nodes,), edge_weight.dtype)])
    deg = jax.ops.segment_sum(w_e, src_e, num_segments=num_nodes)
    dis = jnp.where(deg > 0, lax.rsqrt(deg), 0.0)
    w_norm = -(dis[src_e] * w_e * dis[dst_e])

    ne = src_e.shape[0]
    pad = E_PAD - ne
    srcp = jnp.pad(src_e, (0, pad)).reshape(NT, NB, EB)
    dstp = jnp.pad(dst_e, (0, pad)).reshape(NT, NB, EB)
    wp = jnp.pad(w_norm.astype(jnp.float32), (0, pad)).reshape(NT, NB, EB)

    k = W1.shape[0]
    x_comb = jnp.concatenate([x, lap_pe], axis=1)
    t0_1 = _stack_halves(x_comb, 80)

    tx1, tx2, tx3 = _CHEB[80](t0_1, srcp, dstp, wp)
    w1p = jnp.pad(W1, ((0, 0), (0, 160 - W1.shape[1]), (0, 0)))
    wcat1 = w1p.reshape(k * 160, W1.shape[2])
    h1 = _mm_relu_stacked([t0_1, tx1, tx2, tx3], wcat1, b1.reshape(1, -1))

    ty1, ty2, ty3 = _CHEB[64](h1, srcp, dstp, wp)
    wcat2 = W2.reshape(k * W2.shape[1], W2.shape[2])
    hsum = _mm_relu_sum([h1, ty1, ty2, ty3], wcat2, b2.reshape(1, -1))

    mu, lv = _heads(hsum, Wmu, bmu, Wlv, blv)
    return (mu, lv)


# speedup vs baseline: 2.1081x; 2.1081x over previous
import functools

import jax
import jax.numpy as jnp
from jax import lax
from jax.experimental import pallas as pl
from jax.experimental.pallas import tpu as pltpu
from jax.experimental.pallas import tpu_sc as plsc

N_NODES = 10000
N_ACC = 10240
BLK = 400

NT = 16
EB = 128
NB = 162
EPT = NB * EB
E_PAD = NT * EPT
SB_LEN = 27
RPT = N_ACC // NT
DR = 128

_i32 = jnp.int32


def _make_cheb_layer(DH):
    mesh = plsc.VectorSubcoreMesh(core_axis_name="c", subcore_axis_name="s")
    oshape = jax.ShapeDtypeStruct((2, N_ACC, DH), jnp.float32)

    @functools.partial(
        pl.kernel,
        out_type=(oshape, oshape, oshape, oshape, oshape),
        mesh=mesh,
        compiler_params=pltpu.CompilerParams(use_tc_tiling_on_sc=False),
        scratch_types=[
            pltpu.VMEM((SB_LEN, EB), jnp.int32),
            pltpu.VMEM((SB_LEN, EB), jnp.int32),
            pltpu.VMEM((SB_LEN, EB), jnp.float32),
            pltpu.VMEM((EB, DH), jnp.float32),
            pltpu.VMEM((EB, DH), jnp.float32),
            pltpu.VMEM((DR,), jnp.float32),
            pltpu.VMEM_SHARED((N_ACC, DH), jnp.float32),
            pltpu.SemaphoreType.DMA,
        ],
    )
    def layer(t0_hbm, t0s_hbm, dis_hbm, src_hbm, dst_hbm, w_hbm,
              tx1_hbm, tx2_hbm, tx3_hbm, tx1s_hbm, tx2s_hbm,
              srcv, dstv, wv, rows, rows2, disv, acc, sem):
        c = lax.axis_index("c")
        s = lax.axis_index("s")
        base = s * _i32(RPT)

        def zero_acc():
            def zrow(r, _):
                for ch in range(DH // 16):
                    rows[r, pl.ds(ch * 16, 16)] = jnp.zeros((16,), jnp.float32)
                return 0

            lax.fori_loop(_i32(0), _i32(EB), zrow, 0)
            for k in range(RPT // DR):
                pltpu.sync_copy(rows, acc.at[pl.ds(base + _i32(k * DR), DR)])

        def edge_sweep(ts_hbm):
            def superstep(sb, _):
                off = sb * _i32(SB_LEN)
                pltpu.sync_copy(src_hbm.at[s, pl.ds(off, SB_LEN)], srcv)
                pltpu.sync_copy(dst_hbm.at[s, pl.ds(off, SB_LEN)], dstv)
                pltpu.sync_copy(w_hbm.at[s, pl.ds(off, SB_LEN)], wv)

                def step(g, _):
                    pltpu.async_copy(
                        ts_hbm.at[c].at[srcv.at[g]], rows, sem
                    ).wait()

                    def sgrp(q, _):
                        w16 = wv[g, pl.ds(q * _i32(16), 16)]
                        for jj in range(16):
                            wj = w16[jj]
                            j = q * _i32(16) + _i32(jj)
                            for ch in range(DH // 16):
                                sl = pl.ds(ch * 16, 16)
                                rows[j, sl] = rows[j, sl] * wj
                        return 0

                    lax.fori_loop(_i32(0), _i32(EB // 16), sgrp, 0)
                    pltpu.sync_copy(rows, acc.at[dstv.at[g]], add=True)
                    return 0

                lax.fori_loop(_i32(0), _i32(SB_LEN), step, 0)
                return 0

            lax.fori_loop(_i32(0), _i32(NB // SB_LEN), superstep, 0)

        def drain(out_hbm, outs_hbm, prev_hbm, two):
            for k in range(RPT // DR):
                st = base + _i32(k * DR)
                pltpu.sync_copy(acc.at[pl.ds(st, DR)], rows)
                pltpu.sync_copy(dis_hbm.at[pl.ds(st, DR)], disv)
                if prev_hbm is not None:
                    pltpu.sync_copy(prev_hbm.at[c, pl.ds(st, DR)], rows2)
                scale = -2.0 if two else -1.0

                def dgrp(q, _):
                    d16 = disv[pl.ds(q * _i32(16), 16)]
                    for jj in range(16):
                        dj = d16[jj]
                        r = q * _i32(16) + _i32(jj)
                        for ch in range(DH // 16):
                            sl = pl.ds(ch * 16, 16)
                            y = rows[r, sl] * (scale * dj)
                            if prev_hbm is not None:
                                y = y - rows2[r, sl]
                            rows[r, sl] = y
                            if outs_hbm is not None:
                                rows2[r, sl] = y * dj
                    return 0

                lax.fori_loop(_i32(0), _i32(DR // 16), dgrp, 0)
                pltpu.sync_copy(rows, out_hbm.at[c, pl.ds(st, DR)])
                if outs_hbm is not None:
                    pltpu.sync_copy(rows2, outs_hbm.at[c, pl.ds(st, DR)])

        def mv_pass(ts_hbm, out_hbm, outs_hbm, prev_hbm, two):
            zero_acc()
            plsc.subcore_barrier()
            edge_sweep(ts_hbm)
            plsc.subcore_barrier()
            drain(out_hbm, outs_hbm, prev_hbm, two)
            plsc.subcore_barrier()

        mv_pass(t0s_hbm, tx1_hbm, tx1s_hbm, None, False)
        mv_pass(tx1s_hbm, tx2_hbm, tx2s_hbm, t0_hbm, True)
        mv_pass(tx2s_hbm, tx3_hbm, None, tx1_hbm, True)

    return layer


_CHEB = {dh: _make_cheb_layer(dh) for dh in (80, 64)}


def _cat(ref):
    return jnp.concatenate([ref[0], ref[1]], axis=1)


def _mm_relu_body(t0, t1, t2, t3, w_ref, b_ref, dis_ref, o_ref, os_ref):
    tcat = jnp.concatenate([_cat(t0), _cat(t1), _cat(t2), _cat(t3)], axis=1)
    h = jax.nn.relu(
        jnp.dot(tcat, w_ref[...], preferred_element_type=jnp.float32) + b_ref[...]
    )
    hid2 = o_ref.shape[2]
    o_ref[0] = h[:, :hid2]
    o_ref[1] = h[:, hid2:]
    hs = h * dis_ref[...]
    os_ref[0] = hs[:, :hid2]
    os_ref[1] = hs[:, hid2:]


def _mm_relu_stacked(txs, wcat, b, disc):
    dh = txs[0].shape[2]
    kdim, hid = wcat.shape
    grid = N_NODES // BLK
    tspec = pl.BlockSpec((2, BLK, dh), lambda i: (i * 0, i, i * 0))
    ospec = pl.BlockSpec((2, BLK, hid // 2), lambda i: (i * 0, i, i * 0))
    oshape = jax.ShapeDtypeStruct((2, N_ACC, hid // 2), jnp.float32)
    return pl.pallas_call(
        _mm_relu_body,
        grid=(grid,),
        in_specs=[tspec, tspec, tspec, tspec,
                  pl.BlockSpec((kdim, hid), lambda i: (i * 0, i * 0)),
                  pl.BlockSpec((1, hid), lambda i: (i * 0, i * 0)),
                  pl.BlockSpec((BLK, 1), lambda i: (i, i * 0))],
        out_specs=(ospec, ospec),
        out_shape=(oshape, oshape),
    )(*txs, wcat, b, disc)


def _mm_relu_sum_body(t0, t1, t2, t3, w_ref, b_ref, o_ref):
    i = pl.program_id(0)
    tcat = jnp.concatenate([_cat(t0), _cat(t1), _cat(t2), _cat(t3)], axis=1)
    h = jax.nn.relu(
        jnp.dot(tcat, w_ref[...], preferred_element_type=jnp.float32) + b_ref[...]
    )
    part = jnp.sum(h, axis=0, keepdims=True)

    @pl.when(i == 0)
    def _():
        o_ref[...] = jnp.zeros_like(o_ref)

    o_ref[...] += part


def _mm_relu_sum(txs, wcat, b):
    dh = txs[0].shape[2]
    kdim, hid = wcat.shape
    grid = N_NODES // BLK
    tspec = pl.BlockSpec((2, BLK, dh), lambda i: (i * 0, i, i * 0))
    return pl.pallas_call(
        _mm_relu_sum_body,
        grid=(grid,),
        in_specs=[tspec, tspec, tspec, tspec,
                  pl.BlockSpec((kdim, hid), lambda i: (i * 0, i * 0)),
                  pl.BlockSpec((1, hid), lambda i: (i * 0, i * 0))],
        out_specs=pl.BlockSpec((1, hid), lambda i: (i * 0, i * 0)),
        out_shape=jax.ShapeDtypeStruct((1, hid), jnp.float32),
    )(*txs, wcat, b)


def _heads_body(s_ref, wmu_ref, bmu_ref, wlv_ref, blv_ref, mu_ref, lv_ref):
    ge = s_ref[...] * (1.0 / N_NODES)
    mu_ref[...] = (
        jnp.dot(ge, wmu_ref[...], preferred_element_type=jnp.float32) + bmu_ref[...]
    )
    lv_ref[...] = (
        jnp.dot(ge, wlv_ref[...], preferred_element_type=jnp.float32) + blv_ref[...]
    )


def _heads(hsum, Wmu, bmu, Wlv, blv):
    lat = Wmu.shape[1]
    return pl.pallas_call(
        _heads_body,
        out_shape=(
            jax.ShapeDtypeStruct((1, lat), jnp.float32),
            jax.ShapeDtypeStruct((1, lat), jnp.float32),
        ),
    )(hsum, Wmu, bmu.reshape(1, -1), Wlv, blv.reshape(1, -1))


def _stack_halves(a, dh):
    a = jnp.pad(a, ((0, N_ACC - a.shape[0]), (0, 2 * dh - a.shape[1])))
    return jnp.stack([a[:, :dh], a[:, dh:]])


def kernel(x, edge_index, lap_pe, edge_weight, W1, b1, W2, b2, Wmu, bmu, Wlv, blv):
    num_nodes = x.shape[0]
    src = edge_index[0].astype(jnp.int32)
    dst = edge_index[1].astype(jnp.int32)
    loop = jnp.arange(num_nodes, dtype=jnp.int32)
    src_e = jnp.concatenate([src, loop])
    dst_e = jnp.concatenate([dst, loop])
    w_e = jnp.concatenate([edge_weight, jnp.ones((num_nodes,), edge_weight.dtype)])
    deg = jax.ops.segment_sum(w_e, src_e, num_segments=num_nodes)
    dis = jnp.where(deg > 0, lax.rsqrt(deg), 0.0)
    disp = jnp.pad(dis.astype(jnp.float32), (0, N_ACC - num_nodes))

    ne = src_e.shape[0]
    pad = E_PAD - ne
    srcp = jnp.pad(src_e, (0, pad)).reshape(NT, NB, EB)
    dstp = jnp.pad(dst_e, (0, pad)).reshape(NT, NB, EB)
    wrawp = jnp.pad(w_e.astype(jnp.float32), (0, pad)).reshape(NT, NB, EB)

    k = W1.shape[0]
    x_comb = jnp.concatenate([x, lap_pe], axis=1)
    t0_1 = _stack_halves(x_comb, 80)
    t0s_1 = _stack_halves(x_comb * dis[:, None], 80)

    tx1, tx2, tx3, _, _ = _CHEB[80](t0_1, t0s_1, disp, srcp, dstp, wrawp)
    w1p = jnp.pad(W1, ((0, 0), (0, 160 - W1.shape[1]), (0, 0)))
    wcat1 = w1p.reshape(k * 160, W1.shape[2])
    h1, h1s = _mm_relu_stacked(
        [t0_1, tx1, tx2, tx3], wcat1, b1.reshape(1, -1), disp.reshape(N_ACC, 1)
    )

    ty1, ty2, ty3, _, _ = _CHEB[64](h1, h1s, disp, srcp, dstp, wrawp)
    wcat2 = W2.reshape(k * W2.shape[1], W2.shape[2])
    hsum = _mm_relu_sum([h1, ty1, ty2, ty3], wcat2, b2.reshape(1, -1))

    mu, lv = _heads(hsum, Wmu, bmu, Wlv, blv)
    return (mu, lv)


# speedup vs baseline: 2.7146x; 1.2877x over previous
import functools

import jax
import jax.numpy as jnp
from jax import lax
from jax.experimental import pallas as pl
from jax.experimental.pallas import tpu as pltpu
from jax.experimental.pallas import tpu_sc as plsc

N_NODES = 10000
N_ACC = 10240
BLK = 400

NT = 16
EB = 128
NB = 162
EPT = NB * EB
E_PAD = NT * EPT
SB_LEN = 18
RPT = N_ACC // NT
DR = 128

_i32 = jnp.int32


def _make_cheb_layer(DH):
    mesh = plsc.VectorSubcoreMesh(core_axis_name="c", subcore_axis_name="s")
    oshape = jax.ShapeDtypeStruct((2, N_ACC, DH), jnp.float32)

    @functools.partial(
        pl.kernel,
        out_type=(oshape, oshape, oshape, oshape, oshape),
        mesh=mesh,
        compiler_params=pltpu.CompilerParams(use_tc_tiling_on_sc=False),
        scratch_types=[
            pltpu.VMEM((SB_LEN, EB), jnp.int32),
            pltpu.VMEM((SB_LEN, EB), jnp.int32),
            pltpu.VMEM((SB_LEN, EB), jnp.float32),
            pltpu.VMEM((EB, DH), jnp.float32),
            pltpu.VMEM((EB, DH), jnp.float32),
            pltpu.VMEM((EB, DH), jnp.float32),
            pltpu.VMEM((DR,), jnp.float32),
            pltpu.VMEM_SHARED((N_ACC, DH), jnp.float32),
            pltpu.SemaphoreType.DMA,
            pltpu.SemaphoreType.DMA,
        ],
    )
    def layer(t0_hbm, t0s_hbm, dis_hbm, src_hbm, dst_hbm, w_hbm,
              tx1_hbm, tx2_hbm, tx3_hbm, tx1s_hbm, tx2s_hbm,
              srcv, dstv, wv, rows, rowsb, rows2, disv, acc, sem, semb):
        c = lax.axis_index("c")
        s = lax.axis_index("s")
        base = s * _i32(RPT)

        def zero_acc():
            def zrow(r, _):
                for ch in range(DH // 16):
                    rows[r, pl.ds(ch * 16, 16)] = jnp.zeros((16,), jnp.float32)
                return 0

            lax.fori_loop(_i32(0), _i32(EB), zrow, 0)
            for k in range(RPT // DR):
                pltpu.sync_copy(rows, acc.at[pl.ds(base + _i32(k * DR), DR)])

        def edge_sweep(ts_hbm):
            bufs = (rows, rowsb)
            sems = (sem, semb)

            def scale_scatter(buf, g):
                def sgrp(q, _):
                    w16 = wv[g, pl.ds(q * _i32(16), 16)]
                    for jj in range(16):
                        wj = w16[jj]
                        j = q * _i32(16) + _i32(jj)
                        for ch in range(DH // 16):
                            sl = pl.ds(ch * 16, 16)
                            buf[j, sl] = buf[j, sl] * wj
                    return 0

                lax.fori_loop(_i32(0), _i32(EB // 16), sgrp, 0)
                pltpu.sync_copy(buf, acc.at[dstv.at[g]], add=True)

            def superstep(sb, _):
                off = sb * _i32(SB_LEN)
                pltpu.sync_copy(src_hbm.at[s, pl.ds(off, SB_LEN)], srcv)
                pltpu.sync_copy(dst_hbm.at[s, pl.ds(off, SB_LEN)], dstv)
                pltpu.sync_copy(w_hbm.at[s, pl.ds(off, SB_LEN)], wv)
                pltpu.async_copy(ts_hbm.at[c].at[srcv.at[_i32(0)]], rows, sem)

                def pair(gg, _):
                    g0 = gg * _i32(2)
                    pltpu.make_async_copy(
                        ts_hbm.at[c].at[srcv.at[g0]], rows, sem
                    ).wait()
                    pltpu.async_copy(
                        ts_hbm.at[c].at[srcv.at[g0 + _i32(1)]], rowsb, semb
                    )
                    scale_scatter(rows, g0)
                    pltpu.make_async_copy(
                        ts_hbm.at[c].at[srcv.at[g0 + _i32(1)]], rowsb, semb
                    ).wait()

                    @pl.when(gg < _i32(SB_LEN // 2 - 1))
                    def _():
                        pltpu.async_copy(
                            ts_hbm.at[c].at[srcv.at[g0 + _i32(2)]], rows, sem
                        )

                    scale_scatter(rowsb, g0 + _i32(1))
                    return 0

                lax.fori_loop(_i32(0), _i32(SB_LEN // 2), pair, 0)
                return 0

            lax.fori_loop(_i32(0), _i32(NB // SB_LEN), superstep, 0)

        def drain(out_hbm, outs_hbm, prev_hbm, two):
            for k in range(RPT // DR):
                st = base + _i32(k * DR)
                pltpu.sync_copy(acc.at[pl.ds(st, DR)], rows)
                pltpu.sync_copy(dis_hbm.at[pl.ds(st, DR)], disv)
                if prev_hbm is not None:
                    pltpu.sync_copy(prev_hbm.at[c, pl.ds(st, DR)], rows2)
                scale = -2.0 if two else -1.0

                def dgrp(q, _):
                    d16 = disv[pl.ds(q * _i32(16), 16)]
                    for jj in range(16):
                        dj = d16[jj]
                        r = q * _i32(16) + _i32(jj)
                        for ch in range(DH // 16):
                            sl = pl.ds(ch * 16, 16)
                            y = rows[r, sl] * (scale * dj)
                            if prev_hbm is not None:
                                y = y - rows2[r, sl]
                            rows[r, sl] = y
                            if outs_hbm is not None:
                                rows2[r, sl] = y * dj
                    return 0

                lax.fori_loop(_i32(0), _i32(DR // 16), dgrp, 0)
                pltpu.sync_copy(rows, out_hbm.at[c, pl.ds(st, DR)])
                if outs_hbm is not None:
                    pltpu.sync_copy(rows2, outs_hbm.at[c, pl.ds(st, DR)])

        def mv_pass(ts_hbm, out_hbm, outs_hbm, prev_hbm, two):
            zero_acc()
            plsc.subcore_barrier()
            edge_sweep(ts_hbm)
            plsc.subcore_barrier()
            drain(out_hbm, outs_hbm, prev_hbm, two)
            plsc.subcore_barrier()

        mv_pass(t0s_hbm, tx1_hbm, tx1s_hbm, None, False)
        mv_pass(tx1s_hbm, tx2_hbm, tx2s_hbm, t0_hbm, True)
        mv_pass(tx2s_hbm, tx3_hbm, None, tx1_hbm, True)

    return layer


_CHEB = {dh: _make_cheb_layer(dh) for dh in (80, 64)}


def _cat(ref):
    return jnp.concatenate([ref[0], ref[1]], axis=1)


def _mm_relu_body(t0, t1, t2, t3, w_ref, b_ref, dis_ref, o_ref, os_ref):
    tcat = jnp.concatenate([_cat(t0), _cat(t1), _cat(t2), _cat(t3)], axis=1)
    h = jax.nn.relu(
        jnp.dot(tcat, w_ref[...], preferred_element_type=jnp.float32) + b_ref[...]
    )
    hid2 = o_ref.shape[2]
    o_ref[0] = h[:, :hid2]
    o_ref[1] = h[:, hid2:]
    hs = h * dis_ref[...]
    os_ref[0] = hs[:, :hid2]
    os_ref[1] = hs[:, hid2:]


def _mm_relu_stacked(txs, wcat, b, disc):
    dh = txs[0].shape[2]
    kdim, hid = wcat.shape
    grid = N_NODES // BLK
    tspec = pl.BlockSpec((2, BLK, dh), lambda i: (i * 0, i, i * 0))
    ospec = pl.BlockSpec((2, BLK, hid // 2), lambda i: (i * 0, i, i * 0))
    oshape = jax.ShapeDtypeStruct((2, N_ACC, hid // 2), jnp.float32)
    return pl.pallas_call(
        _mm_relu_body,
        grid=(grid,),
        in_specs=[tspec, tspec, tspec, tspec,
                  pl.BlockSpec((kdim, hid), lambda i: (i * 0, i * 0)),
                  pl.BlockSpec((1, hid), lambda i: (i * 0, i * 0)),
                  pl.BlockSpec((BLK, 1), lambda i: (i, i * 0))],
        out_specs=(ospec, ospec),
        out_shape=(oshape, oshape),
    )(*txs, wcat, b, disc)


def _mm_relu_sum_body(t0, t1, t2, t3, w_ref, b_ref, o_ref):
    i = pl.program_id(0)
    tcat = jnp.concatenate([_cat(t0), _cat(t1), _cat(t2), _cat(t3)], axis=1)
    h = jax.nn.relu(
        jnp.dot(tcat, w_ref[...], preferred_element_type=jnp.float32) + b_ref[...]
    )
    part = jnp.sum(h, axis=0, keepdims=True)

    @pl.when(i == 0)
    def _():
        o_ref[...] = jnp.zeros_like(o_ref)

    o_ref[...] += part


def _mm_relu_sum(txs, wcat, b):
    dh = txs[0].shape[2]
    kdim, hid = wcat.shape
    grid = N_NODES // BLK
    tspec = pl.BlockSpec((2, BLK, dh), lambda i: (i * 0, i, i * 0))
    return pl.pallas_call(
        _mm_relu_sum_body,
        grid=(grid,),
        in_specs=[tspec, tspec, tspec, tspec,
                  pl.BlockSpec((kdim, hid), lambda i: (i * 0, i * 0)),
                  pl.BlockSpec((1, hid), lambda i: (i * 0, i * 0))],
        out_specs=pl.BlockSpec((1, hid), lambda i: (i * 0, i * 0)),
        out_shape=jax.ShapeDtypeStruct((1, hid), jnp.float32),
    )(*txs, wcat, b)


def _heads_body(s_ref, wmu_ref, bmu_ref, wlv_ref, blv_ref, mu_ref, lv_ref):
    ge = s_ref[...] * (1.0 / N_NODES)
    mu_ref[...] = (
        jnp.dot(ge, wmu_ref[...], preferred_element_type=jnp.float32) + bmu_ref[...]
    )
    lv_ref[...] = (
        jnp.dot(ge, wlv_ref[...], preferred_element_type=jnp.float32) + blv_ref[...]
    )


def _heads(hsum, Wmu, bmu, Wlv, blv):
    lat = Wmu.shape[1]
    return pl.pallas_call(
        _heads_body,
        out_shape=(
            jax.ShapeDtypeStruct((1, lat), jnp.float32),
            jax.ShapeDtypeStruct((1, lat), jnp.float32),
        ),
    )(hsum, Wmu, bmu.reshape(1, -1), Wlv, blv.reshape(1, -1))


def _stack_halves(a, dh):
    a = jnp.pad(a, ((0, N_ACC - a.shape[0]), (0, 2 * dh - a.shape[1])))
    return jnp.stack([a[:, :dh], a[:, dh:]])


def kernel(x, edge_index, lap_pe, edge_weight, W1, b1, W2, b2, Wmu, bmu, Wlv, blv):
    num_nodes = x.shape[0]
    src = edge_index[0].astype(jnp.int32)
    dst = edge_index[1].astype(jnp.int32)
    loop = jnp.arange(num_nodes, dtype=jnp.int32)
    src_e = jnp.concatenate([src, loop])
    dst_e = jnp.concatenate([dst, loop])
    w_e = jnp.concatenate([edge_weight, jnp.ones((num_nodes,), edge_weight.dtype)])
    deg = jax.ops.segment_sum(w_e, src_e, num_segments=num_nodes)
    dis = jnp.where(deg > 0, lax.rsqrt(deg), 0.0)
    disp = jnp.pad(dis.astype(jnp.float32), (0, N_ACC - num_nodes))

    ne = src_e.shape[0]
    pad = E_PAD - ne
    srcp = jnp.pad(src_e, (0, pad)).reshape(NT, NB, EB)
    dstp = jnp.pad(dst_e, (0, pad)).reshape(NT, NB, EB)
    wrawp = jnp.pad(w_e.astype(jnp.float32), (0, pad)).reshape(NT, NB, EB)

    k = W1.shape[0]
    x_comb = jnp.concatenate([x, lap_pe], axis=1)
    t0_1 = _stack_halves(x_comb, 80)
    t0s_1 = _stack_halves(x_comb * dis[:, None], 80)

    tx1, tx2, tx3, _, _ = _CHEB[80](t0_1, t0s_1, disp, srcp, dstp, wrawp)
    w1p = jnp.pad(W1, ((0, 0), (0, 160 - W1.shape[1]), (0, 0)))
    wcat1 = w1p.reshape(k * 160, W1.shape[2])
    h1, h1s = _mm_relu_stacked(
        [t0_1, tx1, tx2, tx3], wcat1, b1.reshape(1, -1), disp.reshape(N_ACC, 1)
    )

    ty1, ty2, ty3, _, _ = _CHEB[64](h1, h1s, disp, srcp, dstp, wrawp)
    wcat2 = W2.reshape(k * W2.shape[1], W2.shape[2])
    hsum = _mm_relu_sum([h1, ty1, ty2, ty3], wcat2, b2.reshape(1, -1))

    mu, lv = _heads(hsum, Wmu, bmu, Wlv, blv)
    return (mu, lv)
